# baseline (speedup 1.0000x reference)
import math

import jax
import jax.numpy as jnp
import numpy as np
from jax.experimental import pallas as pl


def _bitrev_perm(n: int) -> np.ndarray:
    log_n = int(math.log2(n))
    perm = np.arange(n).reshape(n, 1)
    for _ in range(log_n):
        n1 = perm.shape[0] // 2
        perm = np.hstack((perm[:n1], perm[n1:]))
    return perm.squeeze(0)


_PERM = _bitrev_perm(32)
_E = np.eye(32, dtype=np.float32)[_PERM]


def _permute_body(x_ref, e_ref, o_ref, *, bm: int):
    x = x_ref[...].reshape(bm * 32, 32)
    et = e_ref[...]
    y = jax.lax.dot(x, et,
                    precision=jax.lax.Precision.HIGHEST,
                    preferred_element_type=jnp.float32)
    yt = jnp.swapaxes(y.reshape(bm, 32, 32), 1, 2).reshape(bm * 32, 32)
    z = jax.lax.dot(yt, et,
                    precision=jax.lax.Precision.HIGHEST,
                    preferred_element_type=jnp.float32)
    o_ref[...] = jnp.swapaxes(z.reshape(bm, 32, 32), 1, 2)


def kernel(tensor):
    n, r, c = tensor.shape
    bm = 256
    body = lambda x_ref, e_ref, o_ref: _permute_body(x_ref, e_ref, o_ref, bm=bm)
    et = jnp.asarray(_E.T)
    return pl.pallas_call(
        body,
        grid=(n // bm,),
        in_specs=[pl.BlockSpec((bm, r, c), lambda i: (i, 0, 0)),
                  pl.BlockSpec((r, c), lambda i: (0, 0))],
        out_specs=pl.BlockSpec((bm, r, c), lambda i: (i, 0, 0)),
        out_shape=jax.ShapeDtypeStruct((n, r, c), tensor.dtype),
    )(tensor, et)


# speedup vs baseline: 2.0359x; 2.0359x over previous
import math

import jax
import jax.numpy as jnp
import numpy as np
from jax.experimental import pallas as pl


def _bitrev_perm(n: int) -> np.ndarray:
    log_n = int(math.log2(n))
    perm = np.arange(n).reshape(n, 1)
    for _ in range(log_n):
        n1 = perm.shape[0] // 2
        perm = np.hstack((perm[:n1], perm[n1:]))
    return perm.squeeze(0)


_PERM = _bitrev_perm(32)
_E = np.eye(32, dtype=np.float32)[_PERM]


def _permute_body(x_ref, e_ref, o_ref, *, bm: int):
    x = x_ref[...].reshape(bm * 32, 32)
    et = e_ref[...]
    y = jax.lax.dot(x, et,
                    preferred_element_type=jnp.float32)
    yt = jnp.swapaxes(y.reshape(bm, 32, 32), 1, 2).reshape(bm * 32, 32)
    z = jax.lax.dot(yt, et,
                    preferred_element_type=jnp.float32)
    o_ref[...] = jnp.swapaxes(z.reshape(bm, 32, 32), 1, 2)


def kernel(tensor):
    n, r, c = tensor.shape
    bm = 256
    body = lambda x_ref, e_ref, o_ref: _permute_body(x_ref, e_ref, o_ref, bm=bm)
    et = jnp.asarray(_E.T)
    return pl.pallas_call(
        body,
        grid=(n // bm,),
        in_specs=[pl.BlockSpec((bm, r, c), lambda i: (i, 0, 0)),
                  pl.BlockSpec((r, c), lambda i: (0, 0))],
        out_specs=pl.BlockSpec((bm, r, c), lambda i: (i, 0, 0)),
        out_shape=jax.ShapeDtypeStruct((n, r, c), tensor.dtype),
    )(tensor, et)


# speedup vs baseline: 4.9894x; 2.4508x over previous
import math

import jax
import jax.numpy as jnp
import numpy as np
from jax.experimental import pallas as pl


def _bitrev_perm(n: int) -> np.ndarray:
    log_n = int(math.log2(n))
    perm = np.arange(n).reshape(n, 1)
    for _ in range(log_n):
        n1 = perm.shape[0] // 2
        perm = np.hstack((perm[:n1], perm[n1:]))
    return perm.squeeze(0)


_PERM = _bitrev_perm(32)
_PERM2 = (_PERM[:, None] * 32 + _PERM[None, :]).reshape(-1)
_G = np.eye(1024, dtype=np.float32)[_PERM2]


def _permute_body(x_ref, g_ref, o_ref):
    x = x_ref[...]
    g = g_ref[...]
    b1 = x.astype(jnp.bfloat16).astype(jnp.float32)
    r1 = x - b1
    b2 = r1.astype(jnp.bfloat16).astype(jnp.float32)
    b3 = r1 - b2
    y = jax.lax.dot(b1, g, preferred_element_type=jnp.float32)
    y += jax.lax.dot(b2, g, preferred_element_type=jnp.float32)
    y += jax.lax.dot(b3, g, preferred_element_type=jnp.float32)
    o_ref[...] = y


def kernel(tensor):
    n, r, c = tensor.shape
    xf = tensor.reshape(n, r * c)
    g = jnp.asarray(_G)
    bm = 512
    out = pl.pallas_call(
        _permute_body,
        grid=(n // bm,),
        in_specs=[pl.BlockSpec((bm, r * c), lambda i: (i, 0)),
                  pl.BlockSpec((r * c, r * c), lambda i: (0, 0))],
        out_specs=pl.BlockSpec((bm, r * c), lambda i: (i, 0)),
        out_shape=jax.ShapeDtypeStruct((n, r * c), tensor.dtype),
    )(xf, g)
    return out.reshape(n, r, c)


# speedup vs baseline: 5.1532x; 1.0328x over previous
import jax
import jax.numpy as jnp
from jax.experimental import pallas as pl
import jax.experimental.pallas.tpu as pltpu


def _swap_lane_bits(v, lam, a, b):
    d = (1 << a) - (1 << b)
    ba = (lam >> a) & 1
    bb = (lam >> b) & 1
    vp = pltpu.roll(v, d, axis=1)
    vm = pltpu.roll(v, 128 - d, axis=1)
    return jnp.where(ba == bb, v, jnp.where(ba == 1, vp, vm))


def _swap_col_lane_bit(lo, hi, lam, lane_bit):
    d = 1 << lane_bit
    bl = (lam >> lane_bit) & 1
    new_lo = jnp.where(bl == 1, pltpu.roll(hi, d, axis=1), lo)
    new_hi = jnp.where(bl == 0, pltpu.roll(lo, 128 - d, axis=1), hi)
    return new_lo, new_hi


def _permute_body(x_ref, o_ref):
    bm = x_ref.shape[0]
    lam = jax.lax.broadcasted_iota(jnp.int32, (bm, 128), 1)
    t = [x_ref[:, 128 * k:128 * (k + 1)] for k in range(8)]
    t = [_swap_lane_bits(v, lam, 4, 0) for v in t]
    t = [_swap_lane_bits(v, lam, 3, 1) for v in t]
    for k in (0, 1, 2, 3):
        t[k], t[k + 4] = _swap_col_lane_bit(t[k], t[k + 4], lam, 5)
    for k in (0, 1, 4, 5):
        t[k], t[k + 2] = _swap_col_lane_bit(t[k], t[k + 2], lam, 6)
    for k in range(8):
        o_ref[:, 128 * k:128 * (k + 1)] = t[k]


def kernel(tensor):
    n, r, c = tensor.shape
    xf = tensor.reshape(n, r * c)
    bm = 512
    out = pl.pallas_call(
        _permute_body,
        grid=(n // bm,),
        in_specs=[pl.BlockSpec((bm, r * c), lambda i: (i, 0))],
        out_specs=pl.BlockSpec((bm, r * c), lambda i: (i, 0)),
        out_shape=jax.ShapeDtypeStruct((n, r * c), tensor.dtype),
    )(xf)
    return out.reshape(n, r, c)
